# Initial kernel scaffold; baseline (speedup 1.0000x reference)
import jax
import jax.numpy as jnp
from jax.experimental import pallas as pl


def kernel(distances, all_i, all_j):
    raise NotImplementedError("write your pallas kernel here")


# speedup vs baseline: 1.5470x; 1.5470x over previous
import functools

import jax
import jax.numpy as jnp
from jax import lax
from jax.experimental import pallas as pl
from jax.experimental.pallas import tpu as pltpu
from jax.experimental.pallas import tpu_sc as plsc

CUTOFF = 5.0
EPSILON = 0.1
SIGMA = 1.0
N_ATOMS = 100000
N_EDGES = 6400000

NC = 2
NS = 16
NW = NC * NS
LANES = 16

EDGES_PER_TILE = N_EDGES // NW
CHUNK = 2000
N_CHUNKS = EDGES_PER_TILE // CHUNK
GROUPS = CHUNK // LANES

NA_PAD = 100096
ATOMS_PER_TILE = NA_PAD // NS

_SHIFT = 4.0 * EPSILON * ((SIGMA / CUTOFF) ** 12 - (SIGMA / CUTOFF) ** 6)
HALF_SHIFT = 0.5 * _SHIFT
TWO_EPS = 2.0 * EPSILON


def _lj_body(dist_hbm, i_hbm, j_hbm, out_hbm,
             dbuf, ibuf, jbuf, vbuf, abuf, accum):
    c = lax.axis_index("c")
    s = lax.axis_index("s")
    wid = s * NC + c

    zero16 = jnp.zeros((LANES,), jnp.float32)

    def zero_body(k, carry):
        abuf[pl.ds(k * LANES, LANES)] = zero16
        return carry

    lax.fori_loop(0, ATOMS_PER_TILE // LANES, zero_body, 0, unroll=8)
    arow = s * ATOMS_PER_TILE
    pltpu.sync_copy(abuf, accum.at[pl.ds(arow, ATOMS_PER_TILE)])
    plsc.subcore_barrier()

    iota = lax.iota(jnp.int32, LANES)

    def chunk_body(g, carry):
        base = wid * EDGES_PER_TILE + g * CHUNK
        pltpu.sync_copy(dist_hbm.at[pl.ds(3 * base, 3 * CHUNK)], dbuf)
        pltpu.sync_copy(i_hbm.at[pl.ds(base, CHUNK)], ibuf)
        pltpu.sync_copy(j_hbm.at[pl.ds(base, CHUNK)], jbuf)

        def vec_body(v, carry2):
            idx0 = v * (3 * LANES) + 3 * iota
            dx = plsc.load_gather(dbuf, [idx0])
            dy = plsc.load_gather(dbuf, [idx0 + 1])
            dz = plsc.load_gather(dbuf, [idx0 + 2])
            r2 = dx * dx + dy * dy + dz * dz
            inv = 1.0 / r2
            s6 = inv * inv * inv
            he = TWO_EPS * (s6 * s6 - s6) - HALF_SHIFT
            vbuf[pl.ds(v * LANES, LANES)] = he
            return carry2

        lax.fori_loop(0, GROUPS, vec_body, 0, unroll=4)

        pltpu.sync_copy(vbuf, accum.at[ibuf], add=True)
        pltpu.sync_copy(vbuf, accum.at[jbuf], add=True)
        return carry

    lax.fori_loop(0, N_CHUNKS, chunk_body, 0)

    plsc.subcore_barrier()
    pltpu.sync_copy(accum.at[pl.ds(arow, ATOMS_PER_TILE)], abuf)
    pltpu.sync_copy(abuf, out_hbm.at[pl.ds(c * NA_PAD + arow, ATOMS_PER_TILE)])


@functools.partial(
    pl.kernel,
    out_type=jax.ShapeDtypeStruct((NC * NA_PAD,), jnp.float32),
    mesh=plsc.VectorSubcoreMesh(core_axis_name="c", subcore_axis_name="s"),
    compiler_params=pltpu.CompilerParams(needs_layout_passes=False),
    scratch_types=[
        pltpu.VMEM((3 * CHUNK,), jnp.float32),
        pltpu.VMEM((CHUNK,), jnp.int32),
        pltpu.VMEM((CHUNK,), jnp.int32),
        pltpu.VMEM((CHUNK,), jnp.float32),
        pltpu.VMEM((ATOMS_PER_TILE,), jnp.float32),
        pltpu.VMEM_SHARED((NA_PAD,), jnp.float32),
    ],
)
def _lj_kernel(dist_hbm, i_hbm, j_hbm, out_hbm,
               dbuf, ibuf, jbuf, vbuf, abuf, accum):
    _lj_body(dist_hbm, i_hbm, j_hbm, out_hbm,
             dbuf, ibuf, jbuf, vbuf, abuf, accum)


def kernel(distances, all_i, all_j):
    dist_flat = distances.reshape(-1)
    partials = _lj_kernel(dist_flat, all_i, all_j)
    partials = partials.reshape(NC, NA_PAD)
    energy = partials[0, :N_ATOMS] + partials[1, :N_ATOMS]
    return energy.reshape(-1, 1)


# speedup vs baseline: 22.5682x; 14.5880x over previous
import functools

import jax
import jax.numpy as jnp
from jax import lax
from jax.experimental import pallas as pl
from jax.experimental.pallas import tpu as pltpu
from jax.experimental.pallas import tpu_sc as plsc

CUTOFF = 5.0
EPSILON = 0.1
SIGMA = 1.0
N_ATOMS = 100000
N_EDGES = 6400000

NC = 2
NS = 16
NW = NC * NS
LANES = 16

CHUNK = 2048
TOTAL_CHUNKS = N_EDGES // CHUNK
MAX_CHUNKS_PER_TILE = -(-TOTAL_CHUNKS // NW)
GROUPS = CHUNK // LANES

NA_PAD = 100096
ATOMS_PER_TILE = NA_PAD // NS

_SHIFT = 4.0 * EPSILON * ((SIGMA / CUTOFF) ** 12 - (SIGMA / CUTOFF) ** 6)
HALF_SHIFT = 0.5 * _SHIFT
TWO_EPS = 2.0 * EPSILON


def _lj_body(dist_hbm, i_hbm, j_hbm, out_hbm,
             dbuf, ibuf, jbuf, vbuf, abuf, accum):
    c = lax.axis_index("c")
    s = lax.axis_index("s")
    wid = s * NC + c

    zero16 = jnp.zeros((LANES,), jnp.float32)

    def zero_body(k, carry):
        abuf[pl.ds(k * LANES, LANES)] = zero16
        return carry

    lax.fori_loop(0, ATOMS_PER_TILE // LANES, zero_body, 0, unroll=8)
    arow = s * ATOMS_PER_TILE
    pltpu.sync_copy(abuf, accum.at[pl.ds(arow, ATOMS_PER_TILE)])
    plsc.subcore_barrier()

    def chunk_body(g, carry):
        cid = g * NW + wid

        @pl.when(cid < TOTAL_CHUNKS)
        def _():
            base = cid * CHUNK
            pltpu.sync_copy(dist_hbm.at[:, pl.ds(base, CHUNK)], dbuf)
            pltpu.sync_copy(i_hbm.at[pl.ds(base, CHUNK)], ibuf)
            pltpu.sync_copy(j_hbm.at[pl.ds(base, CHUNK)], jbuf)

            def vec_body(v, carry2):
                sl = pl.ds(v * LANES, LANES)
                dx = dbuf[0, sl]
                dy = dbuf[1, sl]
                dz = dbuf[2, sl]
                r2 = dx * dx + dy * dy + dz * dz
                inv = 1.0 / r2
                s6 = inv * inv * inv
                he = TWO_EPS * (s6 * s6 - s6) - HALF_SHIFT
                vbuf[sl] = he
                return carry2

            lax.fori_loop(0, GROUPS, vec_body, 0, unroll=4)

            pltpu.sync_copy(vbuf, accum.at[ibuf], add=True)
            pltpu.sync_copy(vbuf, accum.at[jbuf], add=True)

        return carry

    lax.fori_loop(0, MAX_CHUNKS_PER_TILE, chunk_body, 0)

    plsc.subcore_barrier()
    pltpu.sync_copy(accum.at[pl.ds(arow, ATOMS_PER_TILE)], abuf)
    pltpu.sync_copy(abuf, out_hbm.at[pl.ds(c * NA_PAD + arow, ATOMS_PER_TILE)])


@functools.partial(
    pl.kernel,
    out_type=jax.ShapeDtypeStruct((NC * NA_PAD,), jnp.float32),
    mesh=plsc.VectorSubcoreMesh(core_axis_name="c", subcore_axis_name="s"),
    compiler_params=pltpu.CompilerParams(needs_layout_passes=False),
    scratch_types=[
        pltpu.VMEM((3, CHUNK), jnp.float32),
        pltpu.VMEM((CHUNK,), jnp.int32),
        pltpu.VMEM((CHUNK,), jnp.int32),
        pltpu.VMEM((CHUNK,), jnp.float32),
        pltpu.VMEM((ATOMS_PER_TILE,), jnp.float32),
        pltpu.VMEM_SHARED((NA_PAD,), jnp.float32),
    ],
)
def _lj_kernel(dist_hbm, i_hbm, j_hbm, out_hbm,
               dbuf, ibuf, jbuf, vbuf, abuf, accum):
    _lj_body(dist_hbm, i_hbm, j_hbm, out_hbm,
             dbuf, ibuf, jbuf, vbuf, abuf, accum)


def kernel(distances, all_i, all_j):
    dist_t = distances.T
    partials = _lj_kernel(dist_t, all_i, all_j)
    partials = partials.reshape(NC, NA_PAD)
    energy = partials[0, :N_ATOMS] + partials[1, :N_ATOMS]
    return energy.reshape(-1, 1)
